# Initial kernel scaffold; baseline (speedup 1.0000x reference)
import jax
import jax.numpy as jnp
from jax.experimental import pallas as pl


def kernel(x_movie, x_user, edge_index_sims, edge_index_rev, W1_l, W1_r, b1, W2_l, W2_r, b2, W3_l, W3_r, b3, Wlin1, blin1, Wlin2, blin2, Wlin3, blin3):
    raise NotImplementedError("write your pallas kernel here")


# speedup vs baseline: 2.4912x; 2.4912x over previous
import functools

import jax
import jax.numpy as jnp
from jax import lax
from jax.experimental import pallas as pl
from jax.experimental.pallas import tpu as pltpu
from jax.experimental.pallas import tpu_sc as plsc

D = 128
TR = 128


def _make_segsum(e_pad, total_chunks, chunk):
  C = total_chunks // 2
  S = chunk
  S_TR = S + TR
  EPT = e_pad // 16
  NBLK = EPT // 128
  S16 = S // 16
  out_rows = total_chunks * S

  mesh = plsc.VectorSubcoreMesh(core_axis_name="c", subcore_axis_name="s")

  @functools.partial(
      pl.kernel,
      out_type=[
          jax.ShapeDtypeStruct((out_rows, D), jnp.float32),
          jax.ShapeDtypeStruct((out_rows,), jnp.float32),
      ],
      scratch_types=[
          pltpu.VMEM((128,), jnp.int32),
          pltpu.VMEM((128,), jnp.int32),
          pltpu.VMEM((128,), jnp.float32),
          pltpu.VMEM((128, D), jnp.float32),
          pltpu.VMEM((16, D), jnp.float32),
          pltpu.VMEM((1024,), jnp.float32),
          pltpu.VMEM_SHARED((S_TR, D), jnp.float32),
          pltpu.VMEM_SHARED((S_TR,), jnp.float32),
          pltpu.SemaphoreType.DMA,
      ],
      mesh=mesh,
  )
  def k(tab_hbm, src_hbm, dst_hbm, acc_hbm, cnt_hbm,
        srcb, idxb, onesb, rows, zrows, zvec, sh_acc, sh_cnt, sem):
    cid = lax.axis_index("c")
    sid = lax.axis_index("s")
    zeros16 = jnp.zeros((16,), jnp.float32)
    ones16 = jnp.ones((16,), jnp.float32)
    iota16 = lax.iota(jnp.int32, 16)

    def fill_const(i, _):
      onesb[pl.ds(i * 16, 16)] = ones16
      zvec[pl.ds(i * 16, 16)] = zeros16
      return 0
    lax.fori_loop(0, 8, fill_const, 0)

    def fill_z(i, _):
      zvec[pl.ds(128 + i * 16, 16)] = zeros16
      r = i // 8
      zrows[r, pl.ds((i % 8) * 16, 16)] = zeros16
      return 0
    lax.fori_loop(0, 56, fill_z, 0)

    def fill_z2(i, _):
      r = 7 + i // 8
      zrows[r, pl.ds((i % 8) * 16, 16)] = zeros16
      return 0
    lax.fori_loop(0, 72, fill_z2, 0)

    for j in range(C):
      base = (cid * C + j) * S

      def zacc(i, _):
        pltpu.sync_copy(zrows, sh_acc.at[pl.ds(sid * S16 + i * 16, 16)])
        return 0
      lax.fori_loop(0, S16 // 16, zacc, 0)

      @pl.when(sid < 8)
      def _():
        pltpu.sync_copy(zrows, sh_acc.at[pl.ds(S + sid * 16, 16)])

      @pl.when(sid == 8)
      def _():
        def zc(i, _):
          pltpu.sync_copy(zvec, sh_cnt.at[pl.ds(i * 1024, 1024)])
          return 0
        lax.fori_loop(0, S_TR // 1024, zc, 0)
        rem = S_TR % 1024
        if rem:
          pltpu.sync_copy(zvec.at[pl.ds(0, rem)],
                          sh_cnt.at[pl.ds(S_TR - rem, rem)])

      plsc.subcore_barrier()

      def blk(b, _):
        off = sid * EPT + b * 128
        pltpu.sync_copy(src_hbm.at[pl.ds(off, 128)], srcb)
        pltpu.sync_copy(dst_hbm.at[pl.ds(off, 128)], idxb)
        for i in range(8):
          d = idxb[pl.ds(i * 16, 16)]
          loc = d - base
          m = (loc >= 0) & (loc < S)
          tr = (S + ((b * 8 + i) % 8) * 16) + iota16
          idxb[pl.ds(i * 16, 16)] = jnp.where(m, loc, tr)
        pltpu.async_copy(tab_hbm.at[srcb], rows, sem).wait()
        pltpu.sync_copy(rows, sh_acc.at[idxb], add=True)
        pltpu.sync_copy(onesb, sh_cnt.at[idxb], add=True)
        return 0
      lax.fori_loop(0, NBLK, blk, 0)

      plsc.subcore_barrier()

      pltpu.sync_copy(sh_acc.at[pl.ds(sid * S16, S16)],
                      acc_hbm.at[pl.ds(base + sid * S16, S16)])

      @pl.when(sid == 0)
      def _():
        pltpu.sync_copy(sh_cnt.at[pl.ds(0, S)], cnt_hbm.at[pl.ds(base, S)])

      plsc.subcore_barrier()

  return k


def _pad_edges(ei, e_pad, n_src):
  e = ei.shape[1]
  pad = e_pad - e
  psrc = jnp.arange(pad, dtype=jnp.int32) % n_src
  pdst = jnp.full((pad,), -1, jnp.int32)
  src = jnp.concatenate([ei[0], psrc])
  dst = jnp.concatenate([ei[1], pdst])
  return src, dst


def _tc_movie(acc, cnt, x, W1l, W1r, b1, Wl, bl):
  R = 400
  n = 10000

  def body(acc_r, cnt_r, x_r, w1l_r, w1r_r, b1_r, wl_r, bl_r, o_r):
    mean = acc_r[...] / jnp.maximum(cnt_r[...], 1.0)
    t = mean @ w1l_r[...] + x_r[...] @ w1r_r[...] + b1_r[...]
    t = jnp.maximum(t, 0.0)
    h = jnp.maximum(t @ wl_r[...] + bl_r[...], 0.0)
    o_r[...] = h

  full = lambda i: (0, 0)
  return pl.pallas_call(
      body,
      grid=(n // R,),
      in_specs=[
          pl.BlockSpec((R, D), lambda i: (i, 0)),
          pl.BlockSpec((R, 1), lambda i: (i, 0)),
          pl.BlockSpec((R, D), lambda i: (i, 0)),
          pl.BlockSpec((D, D), full),
          pl.BlockSpec((D, D), full),
          pl.BlockSpec((1, D), full),
          pl.BlockSpec((D, D), full),
          pl.BlockSpec((1, D), full),
      ],
      out_specs=pl.BlockSpec((R, D), lambda i: (i, 0)),
      out_shape=jax.ShapeDtypeStruct((n, D), jnp.float32),
  )(acc, cnt, x, W1l, W1r, b1, Wl, bl)


def _tc_user(acc2, cnt2, xu, acc3,
             W2l, W2r, b2, Wl2, bl2, W3l, W3r, b3, Wl3, bl3):
  R = 400
  n = 50000
  OUT = 64

  def body(a2_r, c2_r, xu_r, a3_r,
           w2l_r, w2r_r, b2_r, wl2_r, bl2_r,
           w3l_r, w3r_r, b3_r, wl3_r, bl3_r, o_r):
    rc = 1.0 / jnp.maximum(c2_r[...], 1.0)
    m2 = a2_r[...] * rc
    u1 = jnp.maximum(m2 @ w2l_r[...] + xu_r[...] @ w2r_r[...] + b2_r[...], 0.0)
    u1 = jnp.maximum(u1 @ wl2_r[...] + bl2_r[...], 0.0)
    m3 = a3_r[...] * rc
    u2 = jnp.maximum(m3 @ w3l_r[...] + u1 @ w3r_r[...] + b3_r[...], 0.0)
    o_r[...] = u2 @ wl3_r[...] + bl3_r[...]

  full = lambda i: (0, 0)
  return pl.pallas_call(
      body,
      grid=(n // R,),
      in_specs=[
          pl.BlockSpec((R, D), lambda i: (i, 0)),
          pl.BlockSpec((R, 1), lambda i: (i, 0)),
          pl.BlockSpec((R, D), lambda i: (i, 0)),
          pl.BlockSpec((R, D), lambda i: (i, 0)),
          pl.BlockSpec((D, D), full),
          pl.BlockSpec((D, D), full),
          pl.BlockSpec((1, D), full),
          pl.BlockSpec((D, D), full),
          pl.BlockSpec((1, D), full),
          pl.BlockSpec((D, D), full),
          pl.BlockSpec((D, D), full),
          pl.BlockSpec((1, D), full),
          pl.BlockSpec((D, OUT), full),
          pl.BlockSpec((1, OUT), full),
      ],
      out_specs=pl.BlockSpec((R, OUT), lambda i: (i, 0)),
      out_shape=jax.ShapeDtypeStruct((n, OUT), jnp.float32),
  )(acc2, cnt2, xu, acc3,
    W2l, W2r, b2, Wl2, bl2, W3l, W3r, b3, Wl3, bl3)


_segsum_movie = _make_segsum(e_pad=321536, total_chunks=2, chunk=5120)
_segsum_user = _make_segsum(e_pad=600064, total_chunks=4, chunk=12544)


def kernel(x_movie, x_user, edge_index_sims, edge_index_rev,
           W1_l, W1_r, b1, W2_l, W2_r, b2, W3_l, W3_r, b3,
           Wlin1, blin1, Wlin2, blin2, Wlin3, blin3):
  n_movie = x_movie.shape[0]

  src_s, dst_s = _pad_edges(edge_index_sims, 321536, n_movie)
  src_r, dst_r = _pad_edges(edge_index_rev, 600064, n_movie)

  acc1, cnt1 = _segsum_movie(x_movie, src_s, dst_s)
  h = _tc_movie(acc1, cnt1.reshape(-1, 1), x_movie,
                W1_l, W1_r, b1.reshape(1, -1), Wlin1, blin1.reshape(1, -1))

  acc2, cnt2 = _segsum_user(x_movie, src_r, dst_r)
  acc3, _ = _segsum_user(h, src_r, dst_r)

  return _tc_user(acc2, cnt2.reshape(-1, 1), x_user, acc3,
                  W2_l, W2_r, b2.reshape(1, -1), Wlin2, blin2.reshape(1, -1),
                  W3_l, W3_r, b3.reshape(1, -1), Wlin3, blin3.reshape(1, -1))
